# baseline (speedup 1.0000x reference)
import functools

import jax
import jax.numpy as jnp
import numpy as np
from jax import lax
from jax.experimental import pallas as pl
from jax.experimental.pallas import tpu as pltpu
from jax.experimental.pallas import tpu_sc as plsc

B, L, D = 4096, 200, 64
N_IDX = B * L
NC, NS = 2, 16
NW = NC * NS
W_B = B // NW

MAG = float(np.float32(5.0) / np.sqrt(np.float32(L * D)))

N_ELEM = N_IDX * D
ROWS128 = N_ELEM // 128
BLK = 2048

GB = 2
G_IDX = GB * L
G128 = G_IDX * D // 128
ROWS_PER_B = L * D // 128


def _sc_gather(table, input_ids):
    mesh = plsc.VectorSubcoreMesh(core_axis_name="c", subcore_axis_name="s")

    @functools.partial(
        pl.kernel,
        mesh=mesh,
        compiler_params=pltpu.CompilerParams(use_tc_tiling_on_sc=False),
        out_type=jax.ShapeDtypeStruct((ROWS128, 128), jnp.float32),
        scratch_types=[
            pltpu.VMEM((GB, L), jnp.int32),
            pltpu.VMEM((G_IDX, D), jnp.float32),
            pltpu.VMEM((G128, 128), jnp.float32),
            pltpu.SemaphoreType.DMA,
        ],
    )
    def k(table_hbm, idx_hbm, out_hbm, idx_v, rows_v, rows128_v, sem):
        wid = lax.axis_index("s") * NC + lax.axis_index("c")
        base_b = wid * W_B

        def body(c, _):
            b0 = base_b + c * GB
            pltpu.sync_copy(idx_hbm.at[pl.ds(b0, GB)], idx_v)
            cps = []
            for r in range(GB):
                cps.append(pltpu.async_copy(
                    table_hbm.at[idx_v.at[r, pl.ds(0, 128)]],
                    rows_v.at[pl.ds(r * L, 128)],
                    sem,
                ))
                cps.append(pltpu.async_copy(
                    table_hbm.at[idx_v.at[r, pl.ds(128, L - 128)]],
                    rows_v.at[pl.ds(r * L + 128, L - 128)],
                    sem,
                ))
            for cp in cps:
                cp.wait()

            def repack(q, _):
                for h in range(2):
                    for cc in range(4):
                        rows128_v[q, pl.ds(h * 64 + cc * 16, 16)] = (
                            rows_v[2 * q + h, pl.ds(cc * 16, 16)]
                        )
                return _

            lax.fori_loop(0, G128, repack, None)
            pltpu.sync_copy(rows128_v, out_hbm.at[pl.ds(b0 * ROWS_PER_B, G128)])
            return _

        lax.fori_loop(0, W_B // GB, body, None)

    return k(table, input_ids)


def _sc_add(e128, n128):
    mesh = plsc.VectorSubcoreMesh(core_axis_name="c", subcore_axis_name="s")

    @functools.partial(
        pl.kernel,
        mesh=mesh,
        compiler_params=pltpu.CompilerParams(use_tc_tiling_on_sc=True),
        out_type=jax.ShapeDtypeStruct((B, L, D), jnp.float32),
        scratch_types=[
            pltpu.VMEM((2 * ROWS_PER_B, 128), jnp.float32),
            pltpu.VMEM((2 * ROWS_PER_B, 128), jnp.float32),
            pltpu.VMEM((2, L, D), jnp.float32),
        ],
    )
    def k(e_hbm, n_hbm, out_hbm, e_v, n_v, acc_v, *rest):
        wid = lax.axis_index("s") * NC + lax.axis_index("c")
        base_b = wid * W_B

        def body(c, _):
            b0 = base_b + 2 * c
            r0 = b0 * ROWS_PER_B
            pltpu.sync_copy(e_hbm.at[pl.ds(r0, 2 * ROWS_PER_B)], e_v)
            pltpu.sync_copy(n_hbm.at[pl.ds(r0, 2 * ROWS_PER_B)], n_v)

            def addrow(q, _):
                bb = q // ROWS_PER_B
                ll = 2 * (q % ROWS_PER_B)
                for h in range(2):
                    for cc in range(4):
                        acc_v[bb, ll + h, pl.ds(cc * 16, 16)] = (
                            e_v[q, pl.ds(h * 64 + cc * 16, 16)]
                            + n_v[q, pl.ds(h * 64 + cc * 16, 16)]
                        )
                return _

            lax.fori_loop(0, 2 * ROWS_PER_B, addrow, None)
            pltpu.sync_copy(acc_v, out_hbm.at[pl.ds(b0, 2)])
            return _

        lax.fori_loop(0, W_B // 2, body, None)

    return k(e128, n128)


def _threefry_noise(shape, base):
    it = (
        lax.broadcasted_iota(jnp.int32, shape, 0) * shape[1]
        + lax.broadcasted_iota(jnp.int32, shape, 1)
    ).astype(jnp.uint32)
    x1 = base.astype(jnp.uint32) + it

    k1 = jnp.uint32(42)
    k2 = jnp.uint32(0x1BD11BDA ^ 42)

    def rotl(v, r):
        return (v << jnp.uint32(r)) | (v >> jnp.uint32(32 - r))

    xb = x1 + k1
    xa = xb
    xb = rotl(xb, 13)
    xb = xa ^ xb
    for r in (15, 26, 6):
        xa = xa + xb
        xb = rotl(xb, r)
        xb = xa ^ xb
    xa = xa + k1
    xb = xb + (k2 + jnp.uint32(1))
    ks = (k1, k2, jnp.uint32(0))
    rots = ((17, 29, 16, 24), (13, 15, 26, 6))
    for i in range(1, 5):
        for r in rots[0]:
            xa = xa + xb
            xb = rotl(xb, r)
            xb = xa ^ xb
        xa = xa + ks[1]
        xb = xb + (ks[2] + jnp.uint32(i + 1))
        ks = (ks[1], ks[2], ks[0])
        rots = (rots[1], rots[0])
    bits = xa ^ xb

    uf = lax.bitcast_convert_type(
        (bits >> jnp.uint32(9)) | jnp.uint32(0x3F800000), jnp.float32
    )
    u = uf - jnp.float32(1.0)
    r2 = u * jnp.float32(2.0) - jnp.float32(1.0)
    return r2 * jnp.float32(MAG)


def _noise_body(o_ref):
    pid = pl.program_id(0)
    base = pid * (BLK * 128)
    o_ref[...] = _threefry_noise((BLK, 128), jnp.int32(0) + base)


def _tc_noise():
    return pl.pallas_call(
        _noise_body,
        grid=(ROWS128 // BLK,),
        out_specs=pl.BlockSpec((BLK, 128), lambda i: (i, 0)),
        out_shape=jax.ShapeDtypeStruct((ROWS128, 128), jnp.float32),
    )()


def kernel(input_ids, table):
    n128 = _tc_noise()
    e128 = _sc_gather(table, input_ids.astype(jnp.int32))
    return _sc_add(e128, n128)


# speedup vs baseline: 1.0409x; 1.0409x over previous
import functools

import jax
import jax.numpy as jnp
import numpy as np
from jax import lax
from jax.experimental import pallas as pl
from jax.experimental.pallas import tpu as pltpu
from jax.experimental.pallas import tpu_sc as plsc

B, L, D = 4096, 200, 64
N_IDX = B * L
IDX_COLS = 128
IDX_ROWS = N_IDX // IDX_COLS
NC, NS = 2, 16
NW = NC * NS
W_B = B // NW
W_IDX_ROWS = IDX_ROWS // NW

MAG = float(np.float32(5.0) / np.sqrt(np.float32(L * D)))

N_ELEM = N_IDX * D
ROWS128 = N_ELEM // 128
BLK = 2048
ROWS_PER_B = L * D // 128

CHUNK_IR = 4
CHUNK_ROWS = CHUNK_IR * IDX_COLS
N_CHUNKS = W_IDX_ROWS // CHUNK_IR
C128 = CHUNK_ROWS * D // 128


def _sc_reformat_ids(input_ids):
    mesh = plsc.VectorSubcoreMesh(core_axis_name="c", subcore_axis_name="s")

    @functools.partial(
        pl.kernel,
        mesh=mesh,
        compiler_params=pltpu.CompilerParams(use_tc_tiling_on_sc=True),
        out_type=jax.ShapeDtypeStruct((N_IDX,), jnp.int32),
        scratch_types=[
            pltpu.VMEM((W_B, L), jnp.int32),
            pltpu.VMEM((W_B * L,), jnp.int32),
        ],
    )
    def k(ids_hbm, out_hbm, in_v, out_v):
        wid = lax.axis_index("s") * NC + lax.axis_index("c")
        b0 = wid * W_B
        pltpu.sync_copy(ids_hbm.at[pl.ds(b0, W_B)], in_v)

        def row(r, _):
            base = r * L
            for c in range(12):
                out_v[pl.ds(base + c * 16, 16)] = in_v[r, pl.ds(c * 16, 16)]
            out_v[pl.ds(base + 184, 16)] = in_v[r, pl.ds(184, 16)]
            return _

        lax.fori_loop(0, W_B, row, None)
        pltpu.sync_copy(out_v, out_hbm.at[pl.ds(b0 * L, W_B * L)])

    return k(input_ids)


def _sc_gather(table, idx_flat):
    mesh = plsc.VectorSubcoreMesh(core_axis_name="c", subcore_axis_name="s")

    @functools.partial(
        pl.kernel,
        mesh=mesh,
        compiler_params=pltpu.CompilerParams(use_tc_tiling_on_sc=False),
        out_type=jax.ShapeDtypeStruct((ROWS128, 128), jnp.float32),
        scratch_types=[
            pltpu.VMEM((CHUNK_ROWS,), jnp.int32),
            pltpu.VMEM((CHUNK_ROWS, D), jnp.float32),
            pltpu.VMEM((C128, 128), jnp.float32),
            pltpu.SemaphoreType.DMA,
        ],
    )
    def k(table_hbm, idx_hbm, out_hbm, idx_v, rows_v, rows128_v, sem):
        wid = lax.axis_index("s") * NC + lax.axis_index("c")
        base_ir = wid * W_IDX_ROWS

        def body(c, _):
            ir = base_ir + c * CHUNK_IR
            pltpu.sync_copy(
                idx_hbm.at[pl.ds(ir * IDX_COLS, CHUNK_ROWS)], idx_v
            )
            cps = [
                pltpu.async_copy(
                    table_hbm.at[idx_v.at[pl.ds(j * IDX_COLS, IDX_COLS)]],
                    rows_v.at[pl.ds(j * IDX_COLS, IDX_COLS)],
                    sem,
                )
                for j in range(CHUNK_IR)
            ]
            for cp in cps:
                cp.wait()

            def repack(q, _):
                for h in range(2):
                    for cc in range(4):
                        rows128_v[q, pl.ds(h * 64 + cc * 16, 16)] = (
                            rows_v[2 * q + h, pl.ds(cc * 16, 16)]
                        )
                return _

            lax.fori_loop(0, C128, repack, None)
            pltpu.sync_copy(
                rows128_v, out_hbm.at[pl.ds(ir * IDX_COLS * D // 128, C128)]
            )
            return _

        lax.fori_loop(0, N_CHUNKS, body, None)

    return k(table, idx_flat)


def _sc_add(e128, n128):
    mesh = plsc.VectorSubcoreMesh(core_axis_name="c", subcore_axis_name="s")

    @functools.partial(
        pl.kernel,
        mesh=mesh,
        compiler_params=pltpu.CompilerParams(use_tc_tiling_on_sc=True),
        out_type=jax.ShapeDtypeStruct((B, L, D), jnp.float32),
        scratch_types=[
            pltpu.VMEM((2 * ROWS_PER_B, 128), jnp.float32),
            pltpu.VMEM((2 * ROWS_PER_B, 128), jnp.float32),
            pltpu.VMEM((1, L, D), jnp.float32),
            pltpu.VMEM((1, L, D), jnp.float32),
            pltpu.SemaphoreType.DMA,
            pltpu.SemaphoreType.DMA,
        ],
    )
    def k(e_hbm, n_hbm, out_hbm, e_v, n_v, acc0, acc1, sin, sout):
        wid = lax.axis_index("s") * NC + lax.axis_index("c")
        base_b = wid * W_B
        accs = (acc0, acc1)

        def addhalf(acc, qoff):
            def addrow(q, _):
                ll = 2 * (q - qoff)
                for h in range(2):
                    for cc in range(4):
                        acc[0, ll + h, pl.ds(cc * 16, 16)] = (
                            e_v[q, pl.ds(h * 64 + cc * 16, 16)]
                            + n_v[q, pl.ds(h * 64 + cc * 16, 16)]
                        )
                return _

            lax.fori_loop(qoff, qoff + ROWS_PER_B, addrow, None)

        def body(c, _):
            b0 = base_b + 2 * c
            r0 = b0 * ROWS_PER_B
            cpe = pltpu.async_copy(e_hbm.at[pl.ds(r0, 2 * ROWS_PER_B)], e_v, sin)
            cpn = pltpu.async_copy(n_hbm.at[pl.ds(r0, 2 * ROWS_PER_B)], n_v, sin)
            cpe.wait()
            cpn.wait()
            addhalf(acc0, 0)
            cp0 = pltpu.async_copy(acc0, out_hbm.at[pl.ds(b0, 1)], sout)
            addhalf(acc1, ROWS_PER_B)
            cp1 = pltpu.async_copy(acc1, out_hbm.at[pl.ds(b0 + 1, 1)], sout)
            cp0.wait()
            cp1.wait()
            return _

        lax.fori_loop(0, W_B // 2, body, None)

    return k(e128, n128)


def _threefry_noise(shape, base):
    it = (
        lax.broadcasted_iota(jnp.int32, shape, 0) * shape[1]
        + lax.broadcasted_iota(jnp.int32, shape, 1)
    ).astype(jnp.uint32)
    x1 = base.astype(jnp.uint32) + it

    k1 = jnp.uint32(42)
    k2 = jnp.uint32(0x1BD11BDA ^ 42)

    def rotl(v, r):
        return (v << jnp.uint32(r)) | (v >> jnp.uint32(32 - r))

    xb = x1 + k1
    xa = xb
    xb = rotl(xb, 13)
    xb = xa ^ xb
    for r in (15, 26, 6):
        xa = xa + xb
        xb = rotl(xb, r)
        xb = xa ^ xb
    xa = xa + k1
    xb = xb + (k2 + jnp.uint32(1))
    ks = (k1, k2, jnp.uint32(0))
    rots = ((17, 29, 16, 24), (13, 15, 26, 6))
    for i in range(1, 5):
        for r in rots[0]:
            xa = xa + xb
            xb = rotl(xb, r)
            xb = xa ^ xb
        xa = xa + ks[1]
        xb = xb + (ks[2] + jnp.uint32(i + 1))
        ks = (ks[1], ks[2], ks[0])
        rots = (rots[1], rots[0])
    bits = xa ^ xb

    uf = lax.bitcast_convert_type(
        (bits >> jnp.uint32(9)) | jnp.uint32(0x3F800000), jnp.float32
    )
    u = uf - jnp.float32(1.0)
    r2 = u * jnp.float32(2.0) - jnp.float32(1.0)
    return r2 * jnp.float32(MAG)


def _noise_body(o_ref):
    pid = pl.program_id(0)
    base = pid * (BLK * 128)
    o_ref[...] = _threefry_noise((BLK, 128), jnp.int32(0) + base)


def _tc_noise():
    return pl.pallas_call(
        _noise_body,
        grid=(ROWS128 // BLK,),
        out_specs=pl.BlockSpec((BLK, 128), lambda i: (i, 0)),
        out_shape=jax.ShapeDtypeStruct((ROWS128, 128), jnp.float32),
    )()


def kernel(input_ids, table):
    n128 = _tc_noise()
    ids2d = _sc_reformat_ids(input_ids.astype(jnp.int32))
    e128 = _sc_gather(table, ids2d)
    return _sc_add(e128, n128)
